# Initial kernel scaffold; baseline (speedup 1.0000x reference)
import jax
import jax.numpy as jnp
from jax.experimental import pallas as pl


def kernel(input, target):
    raise NotImplementedError("write your pallas kernel here")


# speedup vs baseline: 1.4126x; 1.4126x over previous
import functools

import jax
import jax.numpy as jnp
from jax.experimental import pallas as pl
from jax.experimental.pallas import tpu as pltpu


def _body(t_ref, x_ref, out_ref, acc_ref, *, bc, n, nblocks):
    j = pl.program_id(0)

    @pl.when(j == 0)
    def _():
        acc_ref[...] = jnp.zeros_like(acc_ref)

    x = x_ref[...]
    m = x.shape[0]
    t = t_ref[...]
    col = j * bc + jax.lax.broadcasted_iota(jnp.int32, (m, bc), 1)
    sp = jnp.maximum(x, 0.0) + jnp.log1p(jnp.exp(-jnp.abs(x)))
    contrib = jnp.where(col < n, sp, 0.0) - jnp.where(col == t, x, 0.0)
    acc_ref[...] += jnp.sum(contrib, axis=1, keepdims=True)

    @pl.when(j == nblocks - 1)
    def _():
        acc = acc_ref[...]
        mask = t == 0
        loss_sum = jnp.sum(jnp.where(mask, 0.0, acc))
        cnt = jnp.sum(mask.astype(jnp.float32))
        out_ref[0, 0] = loss_sum / jnp.maximum(cnt, 1.0)


def kernel(input, target):
    m, n = input.shape
    bc = 2048
    nblocks = pl.cdiv(n, bc)
    t = target.astype(jnp.int32).reshape(m, 1)
    out = pl.pallas_call(
        functools.partial(_body, bc=bc, n=n, nblocks=nblocks),
        grid=(nblocks,),
        in_specs=[
            pl.BlockSpec((m, 1), lambda j: (0, 0)),
            pl.BlockSpec((m, bc), lambda j: (0, j)),
        ],
        out_specs=pl.BlockSpec(
            (1, 1), lambda j: (0, 0), memory_space=pltpu.SMEM
        ),
        out_shape=jax.ShapeDtypeStruct((1, 1), jnp.float32),
        scratch_shapes=[pltpu.VMEM((m, 1), jnp.float32)],
        compiler_params=pltpu.CompilerParams(
            dimension_semantics=("arbitrary",)
        ),
    )(t, input)
    return out[0, 0]


# speedup vs baseline: 1.4815x; 1.0488x over previous
import functools
import math

import jax
import jax.numpy as jnp
from jax.experimental import pallas as pl
from jax.experimental.pallas import tpu as pltpu

_LOG2E = math.log2(math.e)
_LN2 = math.log(2.0)


def _body(t_ref, x_ref, out_ref, *, br, n, nblocks):
    j = pl.program_id(0)

    x = x_ref[...]
    t = t_ref[...]
    a = jax.lax.abs(x)
    e = jnp.exp2(a * (-_LOG2E))
    u = jnp.log2(1.0 + e)
    sp = jnp.maximum(x, 0.0) + _LN2 * u
    col = jax.lax.broadcasted_iota(jnp.int32, (br, n), 1)
    contrib = sp - jnp.where(col == t, x, 0.0)
    rowsum = jnp.sum(contrib, axis=1, keepdims=True)
    good = t != 0
    psum = jnp.sum(jnp.where(good, rowsum, 0.0))
    pcnt = jnp.sum(jnp.where(good, 0.0, 1.0))

    @pl.when(j == 0)
    def _():
        out_ref[0, 0] = 0.0
        out_ref[0, 1] = 0.0

    out_ref[0, 0] += psum
    out_ref[0, 1] += pcnt

    @pl.when(j == nblocks - 1)
    def _():
        out_ref[0, 0] = out_ref[0, 0] / jnp.maximum(out_ref[0, 1], 1.0)


def kernel(input, target):
    m, n = input.shape
    br = 32
    nblocks = m // br
    t = target.astype(jnp.int32).reshape(m, 1)
    out = pl.pallas_call(
        functools.partial(_body, br=br, n=n, nblocks=nblocks),
        grid=(nblocks,),
        in_specs=[
            pl.BlockSpec((br, 1), lambda j: (j, 0)),
            pl.BlockSpec((br, n), lambda j: (j, 0)),
        ],
        out_specs=pl.BlockSpec(
            (1, 2), lambda j: (0, 0), memory_space=pltpu.SMEM
        ),
        out_shape=jax.ShapeDtypeStruct((1, 2), jnp.float32),
        compiler_params=pltpu.CompilerParams(
            dimension_semantics=("arbitrary",)
        ),
    )(t, input)
    return out[0, 0]
